# baseline (speedup 1.0000x reference)
import functools

import jax
import jax.numpy as jnp
from jax.experimental import pallas as pl
from jax.experimental.pallas import tpu as pltpu

_CAP = 256


def _router_body(x_ref, keys_ref, wrf_ref, wt_ref, tid_ref, tidm_ref,
                 gs_ref, ovf_ref, *, V, N, E):
    x = x_ref[...]
    k = keys_ref[...]
    VN = x.shape[0]
    xx = jnp.sum(x * x, axis=-1, keepdims=True)
    kk = jnp.sum(k * k, axis=-1)[None, :]
    xk = jax.lax.dot_general(x, k, (((1,), (1,)), ((), ())),
                             preferred_element_type=jnp.float32)
    d2 = jnp.maximum(xx + kk - 2.0 * xk, 0.0)
    r = jax.lax.dot_general(x, wrf_ref[...], (((1,), (1,)), ((), ())),
                            preferred_element_type=jnp.float32)
    tok = jax.lax.broadcasted_iota(jnp.int32, (VN, V * E), 0)
    col = jax.lax.broadcasted_iota(jnp.int32, (VN, V * E), 1)
    sel = (col // E) == (tok // N)
    rsel = jnp.where(sel, r, 0.0)
    rown = sum(rsel[:, v * E:(v + 1) * E] for v in range(V))
    logits = -d2 + rown

    idx = jax.lax.broadcasted_iota(jnp.int32, logits.shape, 1)
    m1 = jnp.max(logits, axis=-1, keepdims=True)
    a1 = jnp.min(jnp.where(logits == m1, idx, E), axis=-1, keepdims=True)
    mask1 = idx == a1
    l2 = jnp.where(mask1, -jnp.inf, logits)
    m2 = jnp.max(l2, axis=-1, keepdims=True)
    a2 = jnp.min(jnp.where(l2 == m2, idx, E), axis=-1, keepdims=True)
    mask2 = idx == a2
    s = jnp.exp(m2 - m1)
    g1 = 1.0 / (1.0 + s)
    g2 = s / (1.0 + s)
    w = jnp.where(mask1, g1, 0.0) + jnp.where(mask2, g2, 0.0)

    wts = w.T
    wt_ref[...] = wts[:, None, :]

    ind = (w > 0.0).astype(jnp.float32)
    ti = jax.lax.broadcasted_iota(jnp.int32, (VN, VN), 0)
    tj = jax.lax.broadcasted_iota(jnp.int32, (VN, VN), 1)
    tri = (tj < ti).astype(jnp.float32)
    rank = jax.lax.dot_general(tri, ind, (((1,), (0,)), ((), ())),
                               preferred_element_type=jnp.float32,
                               precision=jax.lax.Precision.HIGHEST)
    cnt = jnp.sum(ind, axis=0)

    tvec = jax.lax.broadcasted_iota(jnp.int32, (1, VN), 1).astype(jnp.float32)
    tvecm = tvec - N * jnp.floor(tvec * (1.0 / N))
    tids, tidms, gss = [], [], []
    siota = jax.lax.broadcasted_iota(jnp.int32, (VN, _CAP), 1).astype(jnp.float32)
    for e in range(E):
        pe = ((rank[:, e:e + 1] == siota) & (ind[:, e:e + 1] > 0.0)
              ).astype(jnp.float32)
        tid_e = jax.lax.dot_general(tvec, pe, (((1,), (0,)), ((), ())),
                                    preferred_element_type=jnp.float32,
                                    precision=jax.lax.Precision.HIGHEST)
        tidm_e = jax.lax.dot_general(tvecm, pe, (((1,), (0,)), ((), ())),
                                     preferred_element_type=jnp.float32,
                                     precision=jax.lax.Precision.HIGHEST)
        gs_e = jax.lax.dot_general(wts[e:e + 1], pe,
                                   (((1,), (0,)), ((), ())),
                                   preferred_element_type=jnp.float32,
                                   precision=jax.lax.Precision.HIGHEST)
        tids.append(tid_e)
        tidms.append(tidm_e)
        gss.append(gs_e)
    tid = jnp.concatenate(tids, axis=0)
    tidm = jnp.concatenate(tidms, axis=0)
    gs = jnp.concatenate(gss, axis=0)
    tid_ref[...] = tid.T[None]
    tidm_ref[...] = tidm.T[None]
    gs_ref[...] = gs.T[None]
    ovf_ref[...] = jnp.broadcast_to((cnt > float(_CAP))
                                    .astype(jnp.float32)[:, None, None],
                                    ovf_ref.shape)


def _ffn_body(x_ref, w1_ref, w2_ref, tid_ref, tidm_ref, gs_ref, ovf_ref,
              wt_ref, out_ref, xs_scr, gsc_scr, *, V, N):
    e = pl.program_id(0)
    kblk = pl.program_id(1)

    @pl.when((e == 0) & (kblk == 0))
    def _():
        out_ref[...] = jnp.zeros_like(out_ref)

    ovf = ovf_ref[0, 0, 0] > 0.5
    VN = x_ref.shape[0]

    @pl.when((kblk == 0) & jnp.logical_not(ovf))
    def _():
        tcol = tid_ref[0]
        gcol = gs_ref[0]
        mcol = tidm_ref[0]
        tio = jax.lax.broadcasted_iota(jnp.int32, (_CAP, VN), 1).astype(jnp.float32)
        g = (tcol == tio).astype(jnp.bfloat16)
        xs = jax.lax.dot_general(g, x_ref[...], (((1,), (0,)), ((), ())),
                                 preferred_element_type=jnp.float32)
        xs_scr[...] = xs.astype(jnp.bfloat16)
        nio = jax.lax.broadcasted_iota(jnp.int32, (_CAP, N), 1).astype(jnp.float32)
        gsc_scr[...] = (mcol == nio).astype(jnp.float32) * gcol

    @pl.when(jnp.logical_not(ovf))
    def _():
        h = jax.lax.dot_general(xs_scr[...], w1_ref[0].astype(jnp.bfloat16),
                                (((1,), (0,)), ((), ())),
                                preferred_element_type=jnp.float32)
        h = h * (0.5 + 0.5 * jax.lax.erf(h * 0.7071067811865476))
        y = jax.lax.dot_general(h.astype(jnp.bfloat16),
                                w2_ref[0].astype(jnp.bfloat16),
                                (((1,), (0,)), ((), ())),
                                preferred_element_type=jnp.float32)
        out_ref[...] += jax.lax.dot_general(
            gsc_scr[...], y, (((0,), (0,)), ((), ())),
            preferred_element_type=jnp.float32)

    @pl.when(ovf)
    def _():
        h = jax.lax.dot_general(x_ref[...], w1_ref[0].astype(jnp.bfloat16),
                                (((1,), (0,)), ((), ())),
                                preferred_element_type=jnp.float32)
        h = h * (0.5 + 0.5 * jax.lax.erf(h * 0.7071067811865476))
        y = jax.lax.dot_general(h.astype(jnp.bfloat16),
                                w2_ref[0].astype(jnp.bfloat16),
                                (((1,), (0,)), ((), ())),
                                preferred_element_type=jnp.float32)
        wy = wt_ref[0, 0][:, None] * y
        out_ref[...] += wy.reshape(V, N, -1).sum(axis=0)


@functools.partial(jax.jit, static_argnames=())
def kernel(views, expert_keys, Wr, W1, b1, W2, b2):
    V, N, D = views.shape
    E, _, DFF = W1.shape
    VN = V * N
    x = views.reshape(VN, D)

    wt, tid, tidm, gs, ovf = pl.pallas_call(
        functools.partial(_router_body, V=V, N=N, E=E),
        grid=(1,),
        in_specs=[
            pl.BlockSpec((VN, D), lambda i: (0, 0)),
            pl.BlockSpec((E, D), lambda i: (0, 0)),
            pl.BlockSpec((V * E, D), lambda i: (0, 0)),
        ],
        out_specs=[
            pl.BlockSpec((E, 1, VN), lambda i: (0, 0, 0)),
            pl.BlockSpec((1, _CAP, E), lambda i: (0, 0, 0)),
            pl.BlockSpec((1, _CAP, E), lambda i: (0, 0, 0)),
            pl.BlockSpec((1, _CAP, E), lambda i: (0, 0, 0)),
            pl.BlockSpec((E, 1, 128), lambda i: (0, 0, 0)),
        ],
        out_shape=[
            jax.ShapeDtypeStruct((E, 1, VN), jnp.float32),
            jax.ShapeDtypeStruct((1, _CAP, E), jnp.float32),
            jax.ShapeDtypeStruct((1, _CAP, E), jnp.float32),
            jax.ShapeDtypeStruct((1, _CAP, E), jnp.float32),
            jax.ShapeDtypeStruct((E, 1, 128), jnp.float32),
        ],
    )(x, expert_keys, Wr.reshape(V * E, D))

    tid = tid.reshape(_CAP, E).T.reshape(E, _CAP, 1)
    tidm = tidm.reshape(_CAP, E).T.reshape(E, _CAP, 1)
    gs = gs.reshape(_CAP, E).T.reshape(E, _CAP, 1)

    BD = 1536
    NK = DFF // BD
    out = pl.pallas_call(
        functools.partial(_ffn_body, V=V, N=N),
        grid=(E, NK),
        in_specs=[
            pl.BlockSpec((VN, D), lambda e, k: (0, 0)),
            pl.BlockSpec((1, D, BD), lambda e, k: (e, 0, k)),
            pl.BlockSpec((1, BD, D), lambda e, k: (e, k, 0)),
            pl.BlockSpec((1, _CAP, 1), lambda e, k: (e, 0, 0)),
            pl.BlockSpec((1, _CAP, 1), lambda e, k: (e, 0, 0)),
            pl.BlockSpec((1, _CAP, 1), lambda e, k: (e, 0, 0)),
            pl.BlockSpec((1, 1, 128), lambda e, k: (e, 0, 0)),
            pl.BlockSpec((1, 1, VN), lambda e, k: (e, 0, 0)),
        ],
        out_specs=pl.BlockSpec((N, D), lambda e, k: (0, 0)),
        out_shape=jax.ShapeDtypeStruct((N, D), jnp.float32),
        scratch_shapes=[
            pltpu.VMEM((_CAP, D), jnp.bfloat16),
            pltpu.VMEM((_CAP, N), jnp.float32),
        ],
    )(x.astype(jnp.bfloat16), W1, W2, tid, tidm, gs, ovf, wt)

    return out


# speedup vs baseline: 1.0946x; 1.0946x over previous
import functools

import jax
import jax.numpy as jnp
from jax.experimental import pallas as pl
from jax.experimental.pallas import tpu as pltpu

_CAP = 256


def _router_body(x_ref, keys_ref, wrf_ref, wt_ref, tid_ref, tidm_ref,
                 gs_ref, ovf_ref, *, V, N, E):
    x = x_ref[...]
    k = keys_ref[...]
    VN = x.shape[0]
    xx = jnp.sum(x * x, axis=-1, keepdims=True)
    kk = jnp.sum(k * k, axis=-1)[None, :]
    xk = jax.lax.dot_general(x, k, (((1,), (1,)), ((), ())),
                             preferred_element_type=jnp.float32)
    d2 = jnp.maximum(xx + kk - 2.0 * xk, 0.0)
    r = jax.lax.dot_general(x, wrf_ref[...], (((1,), (1,)), ((), ())),
                            preferred_element_type=jnp.float32)
    tok = jax.lax.broadcasted_iota(jnp.int32, (VN, V * E), 0)
    col = jax.lax.broadcasted_iota(jnp.int32, (VN, V * E), 1)
    sel = (col // E) == (tok // N)
    rsel = jnp.where(sel, r, 0.0)
    rown = sum(rsel[:, v * E:(v + 1) * E] for v in range(V))
    logits = -d2 + rown

    idx = jax.lax.broadcasted_iota(jnp.int32, logits.shape, 1)
    m1 = jnp.max(logits, axis=-1, keepdims=True)
    a1 = jnp.min(jnp.where(logits == m1, idx, E), axis=-1, keepdims=True)
    mask1 = idx == a1
    l2 = jnp.where(mask1, -jnp.inf, logits)
    m2 = jnp.max(l2, axis=-1, keepdims=True)
    a2 = jnp.min(jnp.where(l2 == m2, idx, E), axis=-1, keepdims=True)
    mask2 = idx == a2
    s = jnp.exp(m2 - m1)
    g1 = 1.0 / (1.0 + s)
    g2 = s / (1.0 + s)
    w = jnp.where(mask1, g1, 0.0) + jnp.where(mask2, g2, 0.0)

    wts = w.T
    wt_ref[...] = wts[:, None, :]

    ind = (w > 0.0).astype(jnp.bfloat16)
    ti = jax.lax.broadcasted_iota(jnp.int32, (VN, VN), 0)
    tj = jax.lax.broadcasted_iota(jnp.int32, (VN, VN), 1)
    tri = (tj < ti).astype(jnp.bfloat16)
    rank = jax.lax.dot_general(tri, ind, (((1,), (0,)), ((), ())),
                               preferred_element_type=jnp.float32)
    cnt = jnp.sum(ind.astype(jnp.float32), axis=0)

    tvec = jax.lax.broadcasted_iota(jnp.int32, (1, VN), 1).astype(jnp.float32)
    vvec = jnp.floor(tvec * (1.0 / N)).astype(jnp.bfloat16)
    tvecm = (tvec - N * jnp.floor(tvec * (1.0 / N))).astype(jnp.bfloat16)
    g_hi = wts.astype(jnp.bfloat16)
    g_lo = (wts - g_hi.astype(jnp.float32)).astype(jnp.bfloat16)
    tids, tidms, gss = [], [], []
    siota = jax.lax.broadcasted_iota(jnp.int32, (VN, _CAP), 1).astype(jnp.float32)
    for e in range(E):
        pe = ((rank[:, e:e + 1] == siota) & (ind[:, e:e + 1] > 0)
              ).astype(jnp.bfloat16)
        dn = (((1,), (0,)), ((), ()))
        vv_e = jax.lax.dot_general(vvec, pe, dn,
                                   preferred_element_type=jnp.float32)
        tidm_e = jax.lax.dot_general(tvecm, pe, dn,
                                     preferred_element_type=jnp.float32)
        tid_e = vv_e * N + tidm_e
        gs_e = (jax.lax.dot_general(g_hi[e:e + 1], pe, dn,
                                    preferred_element_type=jnp.float32)
                + jax.lax.dot_general(g_lo[e:e + 1], pe, dn,
                                      preferred_element_type=jnp.float32))
        tids.append(tid_e)
        tidms.append(tidm_e)
        gss.append(gs_e)
    tid = jnp.concatenate(tids, axis=0)
    tidm = jnp.concatenate(tidms, axis=0)
    gs = jnp.concatenate(gss, axis=0)
    tid_ref[...] = tid.T[None]
    tidm_ref[...] = tidm.T[None]
    gs_ref[...] = gs.T[None]
    ovf_ref[...] = jnp.broadcast_to((cnt > float(_CAP))
                                    .astype(jnp.float32)[:, None, None],
                                    ovf_ref.shape)


def _ffn_body(x_ref, w1_ref, w2_ref, tid_ref, tidm_ref, gs_ref, ovf_ref,
              wt_ref, out_ref, xs_scr, gsc_scr, *, V, N):
    e = pl.program_id(0)
    kblk = pl.program_id(1)

    @pl.when((e == 0) & (kblk == 0))
    def _():
        out_ref[...] = jnp.zeros_like(out_ref)

    ovf = ovf_ref[0, 0, 0] > 0.5
    VN = x_ref.shape[0]

    @pl.when((kblk == 0) & jnp.logical_not(ovf))
    def _():
        tcol = tid_ref[0]
        gcol = gs_ref[0]
        mcol = tidm_ref[0]
        tio = jax.lax.broadcasted_iota(jnp.int32, (_CAP, VN), 1).astype(jnp.float32)
        g = (tcol == tio).astype(jnp.bfloat16)
        xs = jax.lax.dot_general(g, x_ref[...], (((1,), (0,)), ((), ())),
                                 preferred_element_type=jnp.float32)
        xs_scr[...] = xs.astype(jnp.bfloat16)
        nio = jax.lax.broadcasted_iota(jnp.int32, (_CAP, N), 1).astype(jnp.float32)
        gsc_scr[...] = (mcol == nio).astype(jnp.float32) * gcol

    @pl.when(jnp.logical_not(ovf))
    def _():
        h = jax.lax.dot_general(xs_scr[...], w1_ref[0].astype(jnp.bfloat16),
                                (((1,), (0,)), ((), ())),
                                preferred_element_type=jnp.float32)
        h = h * (0.5 + 0.5 * jax.lax.erf(h * 0.7071067811865476))
        y = jax.lax.dot_general(h.astype(jnp.bfloat16),
                                w2_ref[0].astype(jnp.bfloat16),
                                (((1,), (0,)), ((), ())),
                                preferred_element_type=jnp.float32)
        out_ref[...] += jax.lax.dot_general(
            gsc_scr[...], y, (((0,), (0,)), ((), ())),
            preferred_element_type=jnp.float32)

    @pl.when(ovf)
    def _():
        h = jax.lax.dot_general(x_ref[...], w1_ref[0].astype(jnp.bfloat16),
                                (((1,), (0,)), ((), ())),
                                preferred_element_type=jnp.float32)
        h = h * (0.5 + 0.5 * jax.lax.erf(h * 0.7071067811865476))
        y = jax.lax.dot_general(h.astype(jnp.bfloat16),
                                w2_ref[0].astype(jnp.bfloat16),
                                (((1,), (0,)), ((), ())),
                                preferred_element_type=jnp.float32)
        wy = wt_ref[0, 0][:, None] * y
        out_ref[...] += wy.reshape(V, N, -1).sum(axis=0)


@functools.partial(jax.jit, static_argnames=())
def kernel(views, expert_keys, Wr, W1, b1, W2, b2):
    V, N, D = views.shape
    E, _, DFF = W1.shape
    VN = V * N
    x = views.reshape(VN, D)

    wt, tid, tidm, gs, ovf = pl.pallas_call(
        functools.partial(_router_body, V=V, N=N, E=E),
        grid=(1,),
        in_specs=[
            pl.BlockSpec((VN, D), lambda i: (0, 0)),
            pl.BlockSpec((E, D), lambda i: (0, 0)),
            pl.BlockSpec((V * E, D), lambda i: (0, 0)),
        ],
        out_specs=[
            pl.BlockSpec((E, 1, VN), lambda i: (0, 0, 0)),
            pl.BlockSpec((1, _CAP, E), lambda i: (0, 0, 0)),
            pl.BlockSpec((1, _CAP, E), lambda i: (0, 0, 0)),
            pl.BlockSpec((1, _CAP, E), lambda i: (0, 0, 0)),
            pl.BlockSpec((E, 1, 128), lambda i: (0, 0, 0)),
        ],
        out_shape=[
            jax.ShapeDtypeStruct((E, 1, VN), jnp.float32),
            jax.ShapeDtypeStruct((1, _CAP, E), jnp.float32),
            jax.ShapeDtypeStruct((1, _CAP, E), jnp.float32),
            jax.ShapeDtypeStruct((1, _CAP, E), jnp.float32),
            jax.ShapeDtypeStruct((E, 1, 128), jnp.float32),
        ],
    )(x, expert_keys, Wr.reshape(V * E, D))

    tid = tid.reshape(_CAP, E).T.reshape(E, _CAP, 1)
    tidm = tidm.reshape(_CAP, E).T.reshape(E, _CAP, 1)
    gs = gs.reshape(_CAP, E).T.reshape(E, _CAP, 1)

    BD = 1536
    NK = DFF // BD
    out = pl.pallas_call(
        functools.partial(_ffn_body, V=V, N=N),
        grid=(E, NK),
        in_specs=[
            pl.BlockSpec((VN, D), lambda e, k: (0, 0)),
            pl.BlockSpec((1, D, BD), lambda e, k: (e, 0, k)),
            pl.BlockSpec((1, BD, D), lambda e, k: (e, k, 0)),
            pl.BlockSpec((1, _CAP, 1), lambda e, k: (e, 0, 0)),
            pl.BlockSpec((1, _CAP, 1), lambda e, k: (e, 0, 0)),
            pl.BlockSpec((1, _CAP, 1), lambda e, k: (e, 0, 0)),
            pl.BlockSpec((1, 1, 128), lambda e, k: (e, 0, 0)),
            pl.BlockSpec((1, 1, VN), lambda e, k: (e, 0, 0)),
        ],
        out_specs=pl.BlockSpec((N, D), lambda e, k: (0, 0)),
        out_shape=jax.ShapeDtypeStruct((N, D), jnp.float32),
        scratch_shapes=[
            pltpu.VMEM((_CAP, D), jnp.bfloat16),
            pltpu.VMEM((_CAP, N), jnp.float32),
        ],
    )(x.astype(jnp.bfloat16), W1, W2, tid, tidm, gs, ovf, wt)

    return out
